# baseline (speedup 1.0000x reference)
import jax
import jax.numpy as jnp
from jax.experimental import pallas as pl
from jax.experimental.pallas import tpu as pltpu

_B = 1024
_BB = 32


def _fwd_kernel(t_sref, sac_sref, somac_sref, x0_ref, noise_ref, xt_ref, tgt_ref):
    b = pl.program_id(0)
    rows = jax.lax.broadcasted_iota(jnp.int32, (_BB, 1, 1, 1), 0)
    sa = jnp.zeros((_BB, 1, 1, 1), jnp.float32)
    so = jnp.zeros((_BB, 1, 1, 1), jnp.float32)
    for i in range(_BB):
        ti = t_sref[b * _BB + i]
        sa = jnp.where(rows == i, sac_sref[ti], sa)
        so = jnp.where(rows == i, somac_sref[ti], so)
    x = x0_ref[...]
    n = noise_ref[...]
    xt_ref[...] = sa * x + so * n
    tgt_ref[...] = sa * n - so * x


def kernel(x0, t, noise, sqrt_alphas_cumprod, sqrt_one_minus_alphas_cumprod):
    t32 = t.astype(jnp.int32)
    shp = x0.shape
    blk = (_BB,) + shp[1:]

    grid_spec = pltpu.PrefetchScalarGridSpec(
        num_scalar_prefetch=3,
        grid=(_B // _BB,),
        in_specs=[
            pl.BlockSpec(blk, lambda b, *_: (b, 0, 0, 0)),
            pl.BlockSpec(blk, lambda b, *_: (b, 0, 0, 0)),
        ],
        out_specs=[
            pl.BlockSpec(blk, lambda b, *_: (b, 0, 0, 0)),
            pl.BlockSpec(blk, lambda b, *_: (b, 0, 0, 0)),
        ],
    )
    xt, tgt = pl.pallas_call(
        _fwd_kernel,
        grid_spec=grid_spec,
        compiler_params=pltpu.CompilerParams(
            dimension_semantics=("parallel",),
        ),
        out_shape=[
            jax.ShapeDtypeStruct(shp, jnp.float32),
            jax.ShapeDtypeStruct(shp, jnp.float32),
        ],
    )(t32, sqrt_alphas_cumprod, sqrt_one_minus_alphas_cumprod, x0, noise)
    return xt, tgt


# speedup vs baseline: 1.8220x; 1.8220x over previous
import jax
import jax.numpy as jnp
from jax.experimental import pallas as pl
from jax.experimental.pallas import tpu as pltpu

_B = 1024
_D = 4 * 64 * 64
_CB = 32
_NBUF = 4
_NCH = _B // _CB


def _fwd_kernel(t_sref, sac_sref, somac_sref, x0_hbm, n_hbm, xt_hbm, tg_hbm,
                xbuf, nbuf, xtbuf, tgbuf, sem_x, sem_n, sem_xt, sem_tg):

    def in_copy_x(c, s):
        return pltpu.make_async_copy(
            x0_hbm.at[pl.ds(c * _CB, _CB)], xbuf.at[s], sem_x.at[s])

    def in_copy_n(c, s):
        return pltpu.make_async_copy(
            n_hbm.at[pl.ds(c * _CB, _CB)], nbuf.at[s], sem_n.at[s])

    def out_copy_xt(c, s):
        return pltpu.make_async_copy(
            xtbuf.at[s], xt_hbm.at[pl.ds(c * _CB, _CB)], sem_xt.at[s])

    def out_copy_tg(c, s):
        return pltpu.make_async_copy(
            tgbuf.at[s], tg_hbm.at[pl.ds(c * _CB, _CB)], sem_tg.at[s])

    for c in range(_NBUF):
        in_copy_x(c, c).start()
        in_copy_n(c, c).start()

    rows = jax.lax.broadcasted_iota(jnp.int32, (_CB, 1), 0)

    def body(c, carry):
        s = jax.lax.rem(c, _NBUF)
        in_copy_x(c, s).wait()
        in_copy_n(c, s).wait()

        @pl.when(c >= _NBUF)
        def _():
            out_copy_xt(c - _NBUF, s).wait()
            out_copy_tg(c - _NBUF, s).wait()

        sa = jnp.zeros((_CB, 1), jnp.float32)
        so = jnp.zeros((_CB, 1), jnp.float32)
        for i in range(_CB):
            ti = t_sref[c * _CB + i]
            sa = jnp.where(rows == i, sac_sref[ti], sa)
            so = jnp.where(rows == i, somac_sref[ti], so)

        x = xbuf[s]
        n = nbuf[s]
        xtbuf[s] = sa * x + so * n
        tgbuf[s] = sa * n - so * x

        out_copy_xt(c, s).start()
        out_copy_tg(c, s).start()

        @pl.when(c + _NBUF < _NCH)
        def _():
            in_copy_x(c + _NBUF, s).start()
            in_copy_n(c + _NBUF, s).start()
        return carry

    jax.lax.fori_loop(0, _NCH, body, 0)

    for k in range(_NBUF):
        c = _NCH - _NBUF + k
        s = c % _NBUF
        out_copy_xt(c, s).wait()
        out_copy_tg(c, s).wait()


def kernel(x0, t, noise, sqrt_alphas_cumprod, sqrt_one_minus_alphas_cumprod):
    x0r = x0.reshape(_B, _D)
    nr = noise.reshape(_B, _D)
    t32 = t.astype(jnp.int32)

    grid_spec = pltpu.PrefetchScalarGridSpec(
        num_scalar_prefetch=3,
        grid=(1,),
        in_specs=[
            pl.BlockSpec(memory_space=pl.ANY),
            pl.BlockSpec(memory_space=pl.ANY),
        ],
        out_specs=[
            pl.BlockSpec(memory_space=pl.ANY),
            pl.BlockSpec(memory_space=pl.ANY),
        ],
        scratch_shapes=[
            pltpu.VMEM((_NBUF, _CB, _D), jnp.float32),
            pltpu.VMEM((_NBUF, _CB, _D), jnp.float32),
            pltpu.VMEM((_NBUF, _CB, _D), jnp.float32),
            pltpu.VMEM((_NBUF, _CB, _D), jnp.float32),
            pltpu.SemaphoreType.DMA((_NBUF,)),
            pltpu.SemaphoreType.DMA((_NBUF,)),
            pltpu.SemaphoreType.DMA((_NBUF,)),
            pltpu.SemaphoreType.DMA((_NBUF,)),
        ],
    )
    xt, tgt = pl.pallas_call(
        _fwd_kernel,
        grid_spec=grid_spec,
        out_shape=[
            jax.ShapeDtypeStruct((_B, _D), jnp.float32),
            jax.ShapeDtypeStruct((_B, _D), jnp.float32),
        ],
    )(t32, sqrt_alphas_cumprod, sqrt_one_minus_alphas_cumprod, x0r, nr)
    return xt.reshape(x0.shape), tgt.reshape(x0.shape)
